# baseline (speedup 1.0000x reference)
import functools

import jax
import jax.numpy as jnp
from jax.experimental import pallas as pl

_N = 1024
_NT = _N // 8
_KNN_K = 5
_ALPHA = 1.05
_W1 = 5.0
_W2 = 3.0
_BIG = 3.0e38


def _body(m2_ref, apt_ref, aa_ref, m1_ref, a2t_ref, l1_ref, knn_ref):
    m2 = m2_ref[0]
    apt = apt_ref[0]
    aa = aa_ref[0]
    m1 = m1_ref[0]
    a2t = a2t_ref[0]

    inner2 = jnp.dot(m2, apt, preferred_element_type=jnp.float32)
    d1t = jnp.dot(m1, a2t, preferred_element_type=jnp.float32)

    aa_row = a2t[4:5, :]

    cm = d1t[0:8, :]
    R = [jnp.full((8, _N), _BIG, jnp.float32) for _ in range(6)]
    for k in range(_NT):
        x = (aa_row + inner2[k * 8:(k + 1) * 8, :]) + aa[k * 8:(k + 1) * 8, :]
        for j in range(5):
            mj = jnp.minimum(R[j], x)
            x = jnp.maximum(R[j], x)
            R[j] = mj
        R[5] = jnp.minimum(R[5], x)
        if k > 0:
            cm = jnp.minimum(cm, d1t[k * 8:(k + 1) * 8, :])

    l1 = jnp.mean(jnp.min(cm, axis=0))
    l1_ref[...] = jnp.full((1, 1, 128), l1, jnp.float32)

    S = jnp.concatenate(R, axis=0)
    row = jax.lax.broadcasted_iota(jnp.int32, (48, _N), 0)
    acc = jnp.zeros((1, _N), jnp.float32)
    for j in range(_KNN_K + 1):
        m = jnp.min(S, axis=0, keepdims=True)
        if j > 0:
            acc = acc + m
        if j < _KNN_K:
            idx = jnp.min(jnp.where(S == m, row, 48), axis=0, keepdims=True)
            S = jnp.where(row == idx, _BIG, S)

    value = acc / jnp.float32(_KNN_K)
    mean = jnp.mean(value)
    std = jnp.sqrt(jnp.sum((value - mean) ** 2) / jnp.float32(_N - 1))
    thr = mean + _ALPHA * std
    w = (value > thr).astype(jnp.float32)
    knn = jnp.mean(value * w)
    knn_ref[...] = jnp.full((1, 1, 128), knn, jnp.float32)


@functools.partial(jax.jit, static_argnames=())
def kernel(adv_pc, ori_pc):
    B = adv_pc.shape[0]
    aa = jnp.sum(adv_pc * adv_pc, axis=-1, keepdims=True)
    oo = jnp.sum(ori_pc * ori_pc, axis=-1, keepdims=True)
    ones = jnp.ones_like(aa)
    zeros = jnp.zeros_like(adv_pc)
    z5 = jnp.concatenate([zeros, zeros[..., :2]], axis=-1)
    m2 = jnp.concatenate([-2.0 * adv_pc, z5], axis=-1)
    ap = jnp.concatenate([adv_pc, z5], axis=-1)
    m1 = jnp.concatenate([-2.0 * ori_pc, oo, ones, zeros], axis=-1)
    a2 = jnp.concatenate([adv_pc, ones, aa, zeros], axis=-1)
    apt = ap.transpose(0, 2, 1)
    a2t = a2.transpose(0, 2, 1)

    l1, knn = pl.pallas_call(
        _body,
        grid=(B,),
        in_specs=[
            pl.BlockSpec((1, _N, 8), lambda b: (b, 0, 0)),
            pl.BlockSpec((1, 8, _N), lambda b: (b, 0, 0)),
            pl.BlockSpec((1, _N, 1), lambda b: (b, 0, 0)),
            pl.BlockSpec((1, _N, 8), lambda b: (b, 0, 0)),
            pl.BlockSpec((1, 8, _N), lambda b: (b, 0, 0)),
        ],
        out_specs=[
            pl.BlockSpec((1, 1, 128), lambda b: (b, 0, 0)),
            pl.BlockSpec((1, 1, 128), lambda b: (b, 0, 0)),
        ],
        out_shape=[
            jax.ShapeDtypeStruct((B, 1, 128), jnp.float32),
            jax.ShapeDtypeStruct((B, 1, 128), jnp.float32),
        ],
    )(m2, apt, aa, m1, a2t)

    chamfer_loss = jnp.mean(l1[:, 0, 0])
    knn_loss = jnp.mean(knn[:, 0, 0])
    return chamfer_loss * _W1 + knn_loss * _W2
